# baseline (speedup 1.0000x reference)
import functools

import jax
from jax import lax
from jax.experimental import pallas as pl
from jax.experimental.pallas import tpu as pltpu
from jax.experimental.pallas import tpu_sc as plsc

_NC = 2
_NS = 16


@functools.partial(jax.jit, static_argnums=(0, 1, 2))
def _broadcast_rows(B, T, H, W):
    nw = _NC * _NS
    rows_w = T // nw
    ch = 64 if rows_w % 64 == 0 else rows_w
    n_chunks = rows_w // ch
    mesh = plsc.VectorSubcoreMesh(
        core_axis_name="c", subcore_axis_name="s",
        num_cores=_NC, num_subcores=_NS,
    )

    @functools.partial(
        pl.kernel,
        mesh=mesh,
        out_type=jax.ShapeDtypeStruct((B, T, H), W.dtype),
        scratch_types=[
            pltpu.SemaphoreType.DMA,
        ],
    )
    def body(w_hbm, out_hbm, ssem):
        wid = lax.axis_index("s") * _NC + lax.axis_index("c")
        base = wid * rows_w
        copies = []
        for b in range(B):
            copies.append(pltpu.async_copy(
                w_hbm.at[pl.ds(base, rows_w)],
                out_hbm.at[b].at[pl.ds(base, rows_w)],
                ssem))
        for cp in copies:
            cp.wait()

    return body(W)


def kernel(X, W, dim):
    B, T = X.shape
    _, H = W.shape
    return _broadcast_rows(B, T, H, W)


# speedup vs baseline: 56.0673x; 56.0673x over previous
import functools

import jax
from jax import lax
from jax.experimental import pallas as pl
from jax.experimental.pallas import tpu as pltpu
from jax.experimental.pallas import tpu_sc as plsc

_NC = 2
_NS = 16


def _chunk_schedule(rows_w):
    if rows_w % 64 == 0 and rows_w >= 128:
        return [8, 56] + [64] * (rows_w // 64 - 1)
    return [rows_w]


@functools.partial(jax.jit, static_argnums=(0, 1, 2))
def _broadcast_rows(B, T, H, W):
    nw = _NC * _NS
    rows_w = T // nw
    chunks = _chunk_schedule(rows_w)
    n_chunks = len(chunks)
    starts = [sum(chunks[:i]) for i in range(n_chunks)]
    buf_rows = max(chunks)
    mesh = plsc.VectorSubcoreMesh(
        core_axis_name="c", subcore_axis_name="s",
        num_cores=_NC, num_subcores=_NS,
    )

    @functools.partial(
        pl.kernel,
        mesh=mesh,
        out_type=jax.ShapeDtypeStruct((B, T, H), W.dtype),
        scratch_types=[
            pltpu.VMEM((buf_rows, H), W.dtype),
            pltpu.VMEM((buf_rows, H), W.dtype),
            pltpu.SemaphoreType.DMA,
            pltpu.SemaphoreType.DMA,
        ],
    )
    def body(w_hbm, out_hbm, buf0, buf1, lsem, ssem):
        wid = lax.axis_index("s") * _NC + lax.axis_index("c")
        base = wid * rows_w
        bufs = (buf0, buf1)
        loads = [None] * n_chunks
        pending = [[], []]
        loads[0] = pltpu.async_copy(
            w_hbm.at[pl.ds(base, chunks[0])],
            bufs[0].at[pl.ds(0, chunks[0])], lsem)
        for i in range(n_chunks):
            nxt = (i + 1) % 2
            if i + 1 < n_chunks:
                for st in pending[nxt]:
                    st.wait()
                pending[nxt] = []
                loads[i + 1] = pltpu.async_copy(
                    w_hbm.at[pl.ds(base + starts[i + 1], chunks[i + 1])],
                    bufs[nxt].at[pl.ds(0, chunks[i + 1])], lsem)
            loads[i].wait()
            for b in range(B):
                pending[i % 2].append(pltpu.async_copy(
                    bufs[i % 2].at[pl.ds(0, chunks[i])],
                    out_hbm.at[b].at[pl.ds(base + starts[i], chunks[i])],
                    ssem))
        for lst in pending:
            for st in lst:
                st.wait()

    return body(W)


def kernel(X, W, dim):
    B, T = X.shape
    _, H = W.shape
    return _broadcast_rows(B, T, H, W)


# speedup vs baseline: 57.2509x; 1.0211x over previous
import functools

import jax
from jax import lax
from jax.experimental import pallas as pl
from jax.experimental.pallas import tpu as pltpu
from jax.experimental.pallas import tpu_sc as plsc

_NC = 2
_NS = 16


def _chunk_schedule(rows_w):
    if rows_w % 64 == 0 and rows_w >= 128:
        return [64] * (rows_w // 64)
    return [rows_w]


@functools.partial(jax.jit, static_argnums=(0, 1, 2))
def _broadcast_rows(B, T, H, W):
    nw = _NC * _NS
    rows_w = T // nw
    chunks = _chunk_schedule(rows_w)
    n_chunks = len(chunks)
    starts = [sum(chunks[:i]) for i in range(n_chunks)]
    buf_rows = max(chunks)
    mesh = plsc.VectorSubcoreMesh(
        core_axis_name="c", subcore_axis_name="s",
        num_cores=_NC, num_subcores=_NS,
    )

    @functools.partial(
        pl.kernel,
        mesh=mesh,
        out_type=jax.ShapeDtypeStruct((B, T, H), W.dtype),
        scratch_types=[
            pltpu.VMEM((buf_rows, H), W.dtype),
            pltpu.VMEM((buf_rows, H), W.dtype),
            pltpu.SemaphoreType.DMA,
            pltpu.SemaphoreType.DMA,
        ],
    )
    def body(w_hbm, out_hbm, buf0, buf1, lsem, ssem):
        wid = lax.axis_index("s") * _NC + lax.axis_index("c")
        base = wid * rows_w
        bufs = (buf0, buf1)
        loads = [None] * n_chunks
        pending = [[], []]
        loads[0] = pltpu.async_copy(
            w_hbm.at[pl.ds(base, chunks[0])],
            bufs[0].at[pl.ds(0, chunks[0])], lsem)
        for i in range(n_chunks):
            nxt = (i + 1) % 2
            if i + 1 < n_chunks:
                for st in pending[nxt]:
                    st.wait()
                pending[nxt] = []
                loads[i + 1] = pltpu.async_copy(
                    w_hbm.at[pl.ds(base + starts[i + 1], chunks[i + 1])],
                    bufs[nxt].at[pl.ds(0, chunks[i + 1])], lsem)
            loads[i].wait()
            for b in range(B):
                pending[i % 2].append(pltpu.async_copy(
                    bufs[i % 2].at[pl.ds(0, chunks[i])],
                    out_hbm.at[b].at[pl.ds(base + starts[i], chunks[i])],
                    ssem))
        for lst in pending:
            for st in lst:
                st.wait()

    return body(W)


def kernel(X, W, dim):
    B, T = X.shape
    _, H = W.shape
    return _broadcast_rows(B, T, H, W)
